# baseline (speedup 1.0000x reference)
import jax
import jax.numpy as jnp
from jax import lax
from jax.experimental import pallas as pl
from jax.experimental.pallas import tpu as pltpu
from jax.experimental.pallas import tpu_sc as plsc

_TIMESTEPS = 1000
_NROWS = 2048
_W = 256
_NWORKERS = 32
_ROWS_PER_W = _NROWS // _NWORKERS
_HOLO = 1024
_LANES = 16
_NG = _W // _LANES


def _build_holo_row(buf, r, hbuf, qsave, lane_f32):
    for g in range(_NG):
        v = buf[r, pl.ds(g * _LANES, _LANES)]
        q0 = (v * 1000.0).astype(jnp.int32) - 1
        q = jnp.where(q0 < 0, _TIMESTEPS - 1, q0)
        valid = v != 0.0
        _, winner = plsc.scan_count(q, mask=valid)
        val1 = jnp.float32(g * _LANES + 1) + lane_f32
        plsc.store_scatter(hbuf, [q], val1, mask=winner)
        qsave[pl.ds(g * _LANES, _LANES)] = q


def _sc_loss_kernel(rec_hbm, tgt_hbm, out_hbm, rbuf, tbuf, hr, ht,
                    qsr, qst, accv, sem_r, sem_t):
    wid = lax.axis_index("c") * 16 + lax.axis_index("s")
    base = wid * _ROWS_PER_W

    cp_r = pltpu.make_async_copy(rec_hbm.at[pl.ds(base, _ROWS_PER_W)],
                                 rbuf, sem_r)
    cp_t = pltpu.make_async_copy(tgt_hbm.at[pl.ds(base, _ROWS_PER_W)],
                                 tbuf, sem_t)
    cp_r.start()
    cp_t.start()

    lane_f32 = lax.iota(jnp.int32, _LANES).astype(jnp.float32)
    zf = jnp.zeros((_LANES,), jnp.float32)

    @plsc.parallel_loop(0, _HOLO // _LANES)
    def _(j):
        hr[pl.ds(j * _LANES, _LANES)] = zf
        ht[pl.ds(j * _LANES, _LANES)] = zf

    cp_r.wait()
    cp_t.wait()

    def row_body(r, accs):
        a0, a1 = accs
        _build_holo_row(rbuf, r, hr, qsr, lane_f32)
        _build_holo_row(tbuf, r, ht, qst, lane_f32)

        for g in range(_NG):
            q = qsr[pl.ds(g * _LANES, _LANES)]
            yv = jnp.float32(g * _LANES) + lane_f32
            e = plsc.load_gather(hr, [q])
            twin = e == yv + 1.0
            h = plsc.load_gather(ht, [q])
            d = yv - jnp.maximum(h - 1.0, 0.0)
            dm = jnp.where(twin, d, 0.0)
            a0 = a0 + dm * dm
            plsc.store_scatter(hr, [q], zf, mask=twin)
            plsc.store_scatter(ht, [q], zf, mask=twin)

        for g in range(_NG):
            q = qst[pl.ds(g * _LANES, _LANES)]
            yv = jnp.float32(g * _LANES) + lane_f32
            e = plsc.load_gather(ht, [q])
            twin = e == yv + 1.0
            dm = jnp.where(twin, yv, 0.0)
            a1 = a1 + dm * dm
            plsc.store_scatter(ht, [q], zf, mask=twin)

        return (a0, a1)

    zero2 = (jnp.zeros((_LANES,), jnp.float32),) * 2
    accs = lax.fori_loop(0, _ROWS_PER_W, row_body, zero2)
    accv[...] = accs[0] + accs[1]
    pltpu.sync_copy(accv, out_hbm.at[wid])


@jax.jit
def kernel(reconstructed_image, target_image):
    rec = jnp.reshape(reconstructed_image, (_NROWS, _W))
    tgt = jnp.reshape(target_image, (_NROWS, _W))

    mesh = plsc.VectorSubcoreMesh(core_axis_name="c", subcore_axis_name="s")
    partials = pl.kernel(
        _sc_loss_kernel,
        mesh=mesh,
        compiler_params=pltpu.CompilerParams(needs_layout_passes=False),
        out_type=jax.ShapeDtypeStruct((_NWORKERS, _LANES), jnp.float32),
        scratch_types=[
            pltpu.VMEM((_ROWS_PER_W, _W), jnp.float32),
            pltpu.VMEM((_ROWS_PER_W, _W), jnp.float32),
            pltpu.VMEM((_HOLO,), jnp.float32),
            pltpu.VMEM((_HOLO,), jnp.float32),
            pltpu.VMEM((_W,), jnp.int32),
            pltpu.VMEM((_W,), jnp.int32),
            pltpu.VMEM((_LANES,), jnp.float32),
            pltpu.SemaphoreType.DMA,
            pltpu.SemaphoreType.DMA,
        ],
    )(rec, tgt)

    denom = jnp.float32(8 * 1 * 256 * _TIMESTEPS)
    return jnp.sum(partials) / denom


# speedup vs baseline: 1.3398x; 1.3398x over previous
import jax
import jax.numpy as jnp
from jax import lax
from jax.experimental import pallas as pl
from jax.experimental.pallas import tpu as pltpu
from jax.experimental.pallas import tpu_sc as plsc

_TIMESTEPS = 1000
_NROWS = 2048
_W = 256
_NWORKERS = 32
_ROWS_PER_W = _NROWS // _NWORKERS
_HOLO = 1024
_LANES = 16


def _build_holo_row(buf, r, hbuf, lane_f32):
    for g in range(_W // _LANES):
        v = buf[r, pl.ds(g * _LANES, _LANES)]
        q0 = (v * 1000.0).astype(jnp.int32) - 1
        q = jnp.where(q0 < 0, _TIMESTEPS - 1, q0)
        valid = v != 0.0
        _, winner = plsc.scan_count(q, mask=valid)
        val = jnp.float32(g * _LANES) + lane_f32
        plsc.store_scatter(hbuf, [q], val, mask=winner)


def _sc_loss_kernel(rec_hbm, tgt_hbm, out_hbm, rbuf, tbuf, hr, ht,
                    accv, sem_r, sem_t):
    wid = lax.axis_index("c") * 16 + lax.axis_index("s")
    base = wid * _ROWS_PER_W

    cp_r = pltpu.make_async_copy(rec_hbm.at[pl.ds(base, _ROWS_PER_W)],
                                 rbuf, sem_r)
    cp_t = pltpu.make_async_copy(tgt_hbm.at[pl.ds(base, _ROWS_PER_W)],
                                 tbuf, sem_t)
    cp_r.start()
    cp_t.start()

    lane_f32 = lax.iota(jnp.int32, _LANES).astype(jnp.float32)
    zf = jnp.zeros((_LANES,), jnp.float32)

    for j in range(_HOLO // _LANES):
        hr[pl.ds(j * _LANES, _LANES)] = zf
        ht[pl.ds(j * _LANES, _LANES)] = zf

    cp_r.wait()
    cp_t.wait()

    def row_body(r, acc):
        _build_holo_row(rbuf, r, hr, lane_f32)
        _build_holo_row(tbuf, r, ht, lane_f32)
        for j in range(_HOLO // _LANES):
            sl = pl.ds(j * _LANES, _LANES)
            d = hr[sl] - ht[sl]
            acc = acc + d * d
            hr[sl] = zf
            ht[sl] = zf
        return acc

    acc = lax.fori_loop(0, _ROWS_PER_W, row_body, jnp.zeros((_LANES,),
                                                            jnp.float32))
    accv[...] = acc
    pltpu.sync_copy(accv, out_hbm.at[wid])


@jax.jit
def kernel(reconstructed_image, target_image):
    rec = jnp.reshape(reconstructed_image, (_NROWS, _W))
    tgt = jnp.reshape(target_image, (_NROWS, _W))

    mesh = plsc.VectorSubcoreMesh(core_axis_name="c", subcore_axis_name="s")
    partials = pl.kernel(
        _sc_loss_kernel,
        mesh=mesh,
        compiler_params=pltpu.CompilerParams(needs_layout_passes=False),
        out_type=jax.ShapeDtypeStruct((_NWORKERS, _LANES), jnp.float32),
        scratch_types=[
            pltpu.VMEM((_ROWS_PER_W, _W), jnp.float32),
            pltpu.VMEM((_ROWS_PER_W, _W), jnp.float32),
            pltpu.VMEM((_HOLO,), jnp.float32),
            pltpu.VMEM((_HOLO,), jnp.float32),
            pltpu.VMEM((_LANES,), jnp.float32),
            pltpu.SemaphoreType.DMA,
            pltpu.SemaphoreType.DMA,
        ],
    )(rec, tgt)

    denom = jnp.float32(8 * 1 * 256 * _TIMESTEPS)
    return jnp.sum(partials) / denom
